# baseline (speedup 1.0000x reference)
import functools

import jax
import jax.numpy as jnp
from jax import lax
from jax.experimental import pallas as pl
from jax.experimental.pallas import tpu as pltpu
from jax.experimental.pallas import tpu_sc as plsc

_NB = 50
_H = 512
_W = 512
_NPIX = _H * _W
_CROWS = 64
_NCHUNK = _H // _CROWS
_NSUB = 8
_HBINS = 64
_HSIZE = 16 * _HBINS
_NTILES = 32
_VPR = _W // 16


def _sc_hist(att_map, img0, nimg):
    ipt = nimg // _NTILES
    mesh = plsc.VectorSubcoreMesh(core_axis_name="c", subcore_axis_name="s")

    @functools.partial(
        pl.kernel,
        mesh=mesh,
        out_type=jax.ShapeDtypeStruct((nimg * _HSIZE,), jnp.float32),
        compiler_params=pltpu.CompilerParams(needs_layout_passes=False),
        scratch_types=[
            pltpu.VMEM((_CROWS, _W), jnp.float32),
            pltpu.VMEM((_CROWS, _W), jnp.float32),
            [pltpu.VMEM((_HSIZE,), jnp.float32) for _ in range(_NSUB)],
            pltpu.SemaphoreType.DMA,
            pltpu.SemaphoreType.DMA,
        ],
    )
    def hist_kernel(att_hbm, his_hbm, buf0, buf1, subs, sem0, sem1):
        wid = lax.axis_index("s") * 2 + lax.axis_index("c")
        lane = lax.broadcasted_iota(jnp.int32, (16,), 0)
        ones = jnp.ones((16,), jnp.float32)
        zeros = jnp.zeros((16,), jnp.float32)
        bufs = (buf0, buf1)
        sems = (sem0, sem1)

        def start_chunk(im, c):
            src = att_hbm.at[im, 0, pl.ds(c * _CROWS, _CROWS), :]
            return pltpu.async_copy(src, bufs[c % 2], sems[c % 2])

        def process_chunk(buf):
            @plsc.parallel_loop(0, _CROWS * 2, unroll=2)
            def _(i):
                r = i >> 1
                base = (i & 1) * (_W // 2)
                xs = [buf[r, pl.ds(base + v * 16, 16)] for v in range(16)]
                fis = [(x * float(_NB)).astype(jnp.int32) * 16 + lane
                       for x in xs]
                for v in range(16):
                    plsc.addupdate_scatter(subs[v % _NSUB], [fis[v]], ones)

        for t in range(ipt):
            im = img0 + wid * ipt + t

            @pl.loop(0, _HSIZE // 16)
            def _(z):
                off = z * 16
                for u in range(_NSUB):
                    subs[u][pl.ds(off, 16)] = zeros

            handles = [None, None]
            handles[0] = start_chunk(im, 0)
            for c in range(_NCHUNK):
                if c + 1 < _NCHUNK:
                    handles[(c + 1) % 2] = start_chunk(im, c + 1)
                handles[c % 2].wait()
                process_chunk(bufs[c % 2])

            @pl.loop(0, _HSIZE // 16)
            def _(z):
                off = z * 16
                acc = subs[0][pl.ds(off, 16)]
                for u in range(1, _NSUB):
                    acc = acc + subs[u][pl.ds(off, 16)]
                subs[0][pl.ds(off, 16)] = acc

            hoff = pl.multiple_of((im - img0) * _HSIZE, _HSIZE)
            pltpu.sync_copy(subs[0], his_hbm.at[pl.ds(hoff, _HSIZE)])

    return hist_kernel(att_map)


def _apply_next(att_ref, his_ref, prev_ref, out_ref):
    del prev_ref
    _apply_body(att_ref, his_ref, out_ref)


def _apply_body(att_ref, his_ref, out_ref):
    att = att_ref[0, 0]
    h2 = his_ref[0]
    counts = jnp.sum(h2, axis=1, keepdims=True)
    iota = lax.broadcasted_iota(jnp.int32, (_HBINS, 1), 0)
    valid = iota < _NB
    counts = jnp.where(valid, counts, -1.0)

    m = jnp.max(counts)
    ind_max = jnp.min(jnp.where(counts == m, iota, _HBINS))
    masked = jnp.where((iota > ind_max) & valid, counts, -1.0)
    m2 = jnp.max(masked)
    ind_sec = jnp.min(jnp.where(masked == m2, iota, _HBINS))

    thr = ind_sec.astype(jnp.float32) / _NB
    mask = att > thr
    area = jnp.sum(mask.astype(jnp.float32))
    value = jnp.maximum(jnp.sqrt(jnp.sqrt(area)), 1.0)
    inv = 1.0 / value
    supp = jnp.exp(jnp.log(jnp.maximum(att, 1e-6)) * inv)
    out_ref[0, 0] = jnp.where(mask, supp, att)


def kernel(att_map):
    B = att_map.shape[0]
    nslice = 2
    ns = B // nslice
    out = None
    for s in range(nslice):
        img0 = s * ns
        his = _sc_hist(att_map, img0, ns).reshape(ns, _HBINS, 16)
        in_specs = [
            pl.BlockSpec((1, 1, _H, _W),
                         lambda i, o=img0: (i + o, 0, 0, 0)),
            pl.BlockSpec((1, _HBINS, 16), lambda i: (i, 0, 0)),
        ]
        inputs = [att_map, his]
        aliases = {}
        body = _apply_body
        if out is not None:
            in_specs.append(pl.BlockSpec(memory_space=pl.ANY))
            inputs.append(out)
            aliases = {2: 0}
            body = _apply_next
        out = pl.pallas_call(
            body,
            grid=(ns,),
            in_specs=in_specs,
            out_specs=pl.BlockSpec((1, 1, _H, _W),
                                   lambda i, o=img0: (i + o, 0, 0, 0)),
            out_shape=jax.ShapeDtypeStruct((B, 1, _H, _W), jnp.float32),
            input_output_aliases=aliases,
            compiler_params=pltpu.CompilerParams(
                dimension_semantics=("parallel",)),
        )(*inputs)
    return jax.lax.stop_gradient(out)


# speedup vs baseline: 1.0049x; 1.0049x over previous
import functools

import jax
import jax.numpy as jnp
from jax import lax
from jax.experimental import pallas as pl
from jax.experimental.pallas import tpu as pltpu
from jax.experimental.pallas import tpu_sc as plsc

_NB = 50
_H = 512
_W = 512
_NPIX = _H * _W
_CROWS = 64
_NCHUNK = _H // _CROWS
_NSUB = 8
_HBINS = 64
_HSIZE = 16 * _HBINS
_NTILES = 32
_VPR = _W // 16


def _sc_hist(att_map, img0, nimg):
    if nimg >= _NTILES:
        ipt, tpi = nimg // _NTILES, 1
    else:
        ipt, tpi = 1, _NTILES // nimg
    rows = _H // tpi
    nch = rows // _CROWS
    mesh = plsc.VectorSubcoreMesh(core_axis_name="c", subcore_axis_name="s")

    @functools.partial(
        pl.kernel,
        mesh=mesh,
        out_type=jax.ShapeDtypeStruct((nimg * tpi * _HSIZE,), jnp.float32),
        compiler_params=pltpu.CompilerParams(needs_layout_passes=False),
        scratch_types=[
            pltpu.VMEM((_CROWS, _W), jnp.float32),
            pltpu.VMEM((_CROWS, _W), jnp.float32),
            [pltpu.VMEM((_HSIZE,), jnp.float32) for _ in range(_NSUB)],
            pltpu.SemaphoreType.DMA,
            pltpu.SemaphoreType.DMA,
        ],
    )
    def hist_kernel(att_hbm, his_hbm, buf0, buf1, subs, sem0, sem1):
        wid = lax.axis_index("s") * 2 + lax.axis_index("c")
        lane = lax.broadcasted_iota(jnp.int32, (16,), 0)
        ones = jnp.ones((16,), jnp.float32)
        zeros = jnp.zeros((16,), jnp.float32)
        bufs = (buf0, buf1)
        sems = (sem0, sem1)

        def start_chunk(im, r0, c):
            src = att_hbm.at[im, 0, pl.ds(r0 + c * _CROWS, _CROWS), :]
            return pltpu.async_copy(src, bufs[c % 2], sems[c % 2])

        def process_chunk(buf):
            @plsc.parallel_loop(0, _CROWS * 2, unroll=2)
            def _(i):
                r = i >> 1
                base = (i & 1) * (_W // 2)
                xs = [buf[r, pl.ds(base + v * 16, 16)] for v in range(16)]
                fis = [(x * float(_NB)).astype(jnp.int32) * 16 + lane
                       for x in xs]
                for v in range(16):
                    plsc.addupdate_scatter(subs[v % _NSUB], [fis[v]], ones)

        for t in range(ipt):
            if tpi == 1:
                im = img0 + wid * ipt + t
                r0 = 0
                slot = wid * ipt + t
            else:
                im = img0 + wid // tpi
                r0 = (wid % tpi) * rows
                slot = wid

            @pl.loop(0, _HSIZE // 16)
            def _(z):
                off = z * 16
                for u in range(_NSUB):
                    subs[u][pl.ds(off, 16)] = zeros

            handles = [None, None]
            handles[0] = start_chunk(im, r0, 0)
            for c in range(nch):
                if c + 1 < nch:
                    handles[(c + 1) % 2] = start_chunk(im, r0, c + 1)
                handles[c % 2].wait()
                process_chunk(bufs[c % 2])

            @pl.loop(0, _HSIZE // 16)
            def _(z):
                off = z * 16
                acc = subs[0][pl.ds(off, 16)]
                for u in range(1, _NSUB):
                    acc = acc + subs[u][pl.ds(off, 16)]
                subs[0][pl.ds(off, 16)] = acc

            hoff = pl.multiple_of(slot * _HSIZE, _HSIZE)
            pltpu.sync_copy(subs[0], his_hbm.at[pl.ds(hoff, _HSIZE)])

    return hist_kernel(att_map)


def _make_apply(tpi, with_prev):
    def body(att_ref, his_ref, *rest):
        out_ref = rest[-1]
        _apply_body(att_ref, his_ref, out_ref, tpi)
    return body


def _apply_body(att_ref, his_ref, out_ref, tpi):
    att = att_ref[0, 0]
    h2 = his_ref[0]
    acc = h2[0:_HBINS]
    for p in range(1, tpi):
        acc = acc + h2[p * _HBINS:(p + 1) * _HBINS]
    counts = jnp.sum(acc, axis=1, keepdims=True)
    iota = lax.broadcasted_iota(jnp.int32, (_HBINS, 1), 0)
    valid = iota < _NB
    counts = jnp.where(valid, counts, -1.0)

    m = jnp.max(counts)
    ind_max = jnp.min(jnp.where(counts == m, iota, _HBINS))
    masked = jnp.where((iota > ind_max) & valid, counts, -1.0)
    m2 = jnp.max(masked)
    ind_sec = jnp.min(jnp.where(masked == m2, iota, _HBINS))

    thr = ind_sec.astype(jnp.float32) / _NB
    mask = att > thr
    area = jnp.sum(mask.astype(jnp.float32))
    value = jnp.maximum(jnp.sqrt(jnp.sqrt(area)), 1.0)
    inv = 1.0 / value
    supp = jnp.exp(jnp.log(jnp.maximum(att, 1e-6)) * inv)
    out_ref[0, 0] = jnp.where(mask, supp, att)


def kernel(att_map):
    B = att_map.shape[0]
    nslice = 4
    ns = B // nslice
    tpi = max(1, _NTILES // ns)
    out = None
    for s in range(nslice):
        img0 = s * ns
        his = _sc_hist(att_map, img0, ns).reshape(ns, tpi * _HBINS, 16)
        in_specs = [
            pl.BlockSpec((1, 1, _H, _W),
                         lambda i, o=img0: (i + o, 0, 0, 0)),
            pl.BlockSpec((1, tpi * _HBINS, 16), lambda i: (i, 0, 0)),
        ]
        inputs = [att_map, his]
        aliases = {}
        body = _make_apply(tpi, False)
        if out is not None:
            in_specs.append(pl.BlockSpec(memory_space=pl.ANY))
            inputs.append(out)
            aliases = {2: 0}
            body = _make_apply(tpi, True)
        out = pl.pallas_call(
            body,
            grid=(ns,),
            in_specs=in_specs,
            out_specs=pl.BlockSpec((1, 1, _H, _W),
                                   lambda i, o=img0: (i + o, 0, 0, 0)),
            out_shape=jax.ShapeDtypeStruct((B, 1, _H, _W), jnp.float32),
            input_output_aliases=aliases,
            compiler_params=pltpu.CompilerParams(
                dimension_semantics=("parallel",)),
        )(*inputs)
    return jax.lax.stop_gradient(out)
